# baseline (speedup 1.0000x reference)
import functools

import jax
import jax.numpy as jnp
from jax.experimental import pallas as pl
from jax.experimental.pallas import tpu as pltpu

_HIDDEN = 768
_INTER = 2048
_BM = 128
_KI = 2


def _gemm_body(be_ref, nact_ref, x_ref, wg_ref, wu_ref, down_ref, pw_ref,
               o_ref):
    g = pl.program_id(0)
    ki = pl.program_id(1)

    @pl.when(g < nact_ref[0])
    def _():
        x = x_ref[...]
        gate = jax.lax.dot_general(
            x, wg_ref[0], (((1,), (1,)), ((), ())),
            preferred_element_type=jnp.float32)
        up = jax.lax.dot_general(
            x, wu_ref[0], (((1,), (1,)), ((), ())),
            preferred_element_type=jnp.float32)
        act = gate * jax.nn.sigmoid(gate) * up
        dn = down_ref[0]
        o = jax.lax.dot_general(
            act, dn, (((1,), (1,)), ((), ())),
            preferred_element_type=jnp.float32)
        o = o * pw_ref[...]

        @pl.when(ki == 0)
        def _():
            o_ref[...] = o

        @pl.when(ki > 0)
        def _():
            o_ref[...] += o


def _grouped_gemm(x_padded, w13, down_proj, pw_padded, block_expert,
                  num_active, interpret=False):
    G = block_expert.shape[0]
    H, I = _HIDDEN, _INTER
    IC = I // _KI
    grid_spec = pltpu.PrefetchScalarGridSpec(
        num_scalar_prefetch=2,
        grid=(G, _KI),
        in_specs=[
            pl.BlockSpec((_BM, H), lambda g, ki, be, na: (g, 0)),
            pl.BlockSpec((1, IC, H), lambda g, ki, be, na: (be[g], ki, 0)),
            pl.BlockSpec((1, IC, H),
                         lambda g, ki, be, na: (be[g], _KI + ki, 0)),
            pl.BlockSpec((1, H, IC), lambda g, ki, be, na: (be[g], 0, ki)),
            pl.BlockSpec((_BM, 1), lambda g, ki, be, na: (g, 0)),
        ],
        out_specs=pl.BlockSpec((_BM, H), lambda g, ki, be, na: (g, 0)),
    )
    return pl.pallas_call(
        _gemm_body,
        grid_spec=grid_spec,
        out_shape=jax.ShapeDtypeStruct((G * _BM, H), jnp.float32),
        interpret=interpret,
    )(block_expert, num_active, x_padded, w13, w13, down_proj, pw_padded)


@functools.partial(jax.jit, static_argnames=("interpret",))
def _moe(x, expert_indices, expert_weights, w13, down_proj, interpret=False):
    B, S, H = x.shape
    E = w13.shape[0]
    top_k = expert_indices.shape[-1]
    N = B * S
    P = N * top_k
    G = P // _BM + E
    P_pad = G * _BM

    x_flat = x.reshape(N, H)
    flat_e = expert_indices.reshape(P)
    w_flat = expert_weights.reshape(P)

    order = jnp.argsort(flat_e)
    e_sorted = flat_e[order]
    sizes = jnp.bincount(flat_e, length=E)
    blocks_e = (sizes + _BM - 1) // _BM
    starts_unpad = jnp.cumsum(sizes) - sizes
    block_start_e = jnp.cumsum(blocks_e) - blocks_e
    starts_pad = block_start_e * _BM
    ranks = jnp.arange(P, dtype=jnp.int32) - starts_unpad[e_sorted]
    pos_sorted = (starts_pad[e_sorted] + ranks).astype(jnp.int32)

    tok_padded = jnp.zeros((P_pad,), jnp.int32).at[pos_sorted].set(
        (order // top_k).astype(jnp.int32))
    pw_padded = jnp.zeros((P_pad, 1), jnp.float32).at[pos_sorted, 0].set(
        w_flat[order])
    pos_by_pair = jnp.zeros((P,), jnp.int32).at[order].set(pos_sorted)

    num_active = jnp.sum(blocks_e).astype(jnp.int32).reshape(1)
    block_expert = jnp.minimum(
        jnp.searchsorted(jnp.cumsum(blocks_e), jnp.arange(G), side="right"),
        E - 1).astype(jnp.int32)

    x_padded = x_flat[tok_padded]

    o_padded = _grouped_gemm(x_padded, w13, down_proj, pw_padded,
                             block_expert, num_active, interpret=interpret)

    pos2 = pos_by_pair.reshape(N, top_k)
    out = jnp.sum(o_padded[pos2], axis=1)
    return out.reshape(B, S, H)


def kernel(x, expert_indices, expert_weights, w13, down_proj):
    return _moe(x, expert_indices, expert_weights, w13, down_proj)


# speedup vs baseline: 1.5010x; 1.5010x over previous
import functools

import jax
import jax.numpy as jnp
from jax import lax
from jax.experimental import pallas as pl
from jax.experimental.pallas import tpu as pltpu
from jax.experimental.pallas import tpu_sc as plsc

_HIDDEN = 768
_INTER = 2048
_BM = 128
_NC = 2
_NS = 16
_NW = _NC * _NS
_L = 16


def _sc_dispatch(x_flat, tok_sorted, pos_sorted, p_pad):
    N, H = x_flat.shape
    P = tok_sorted.shape[0]
    per_w = P // _NW
    mesh = plsc.VectorSubcoreMesh(core_axis_name="c", subcore_axis_name="s")

    @functools.partial(
        pl.kernel, mesh=mesh,
        out_type=jax.ShapeDtypeStruct((p_pad, H), jnp.float32),
        scratch_types=[
            pltpu.VMEM((per_w,), jnp.int32),
            pltpu.VMEM((per_w,), jnp.int32),
            pltpu.VMEM((per_w, H), jnp.float32),
            pltpu.SemaphoreType.DMA,
            pltpu.SemaphoreType.DMA,
        ],
    )
    def k(x_hbm, tok_hbm, pos_hbm, out_hbm, tok_v, pos_v, rows_v, sem_g,
          sem_s):
        wid = lax.axis_index("s") * _NC + lax.axis_index("c")
        base = wid * per_w
        pltpu.sync_copy(tok_hbm.at[pl.ds(base, per_w)], tok_v)
        pltpu.sync_copy(pos_hbm.at[pl.ds(base, per_w)], pos_v)
        pltpu.async_copy(x_hbm.at[tok_v], rows_v, sem_g).wait()
        pltpu.async_copy(rows_v, out_hbm.at[pos_v], sem_s).wait()

    return k(x_flat, tok_sorted, pos_sorted)


def _sc_combine(o_padded, pos_pair, n_tokens):
    H = o_padded.shape[1]
    tok_w = n_tokens // _NW
    pair_w = 2 * tok_w
    vpr = H // _L
    mesh = plsc.VectorSubcoreMesh(core_axis_name="c", subcore_axis_name="s")

    @functools.partial(
        pl.kernel, mesh=mesh,
        out_type=jax.ShapeDtypeStruct((n_tokens, H), jnp.float32),
        scratch_types=[
            pltpu.VMEM((pair_w,), jnp.int32),
            pltpu.VMEM((pair_w, H), jnp.float32),
            pltpu.SemaphoreType.DMA,
        ],
    )
    def k(o_hbm, pos_hbm, out_hbm, idx_v, rows_v, sem):
        wid = lax.axis_index("s") * _NC + lax.axis_index("c")
        pltpu.sync_copy(pos_hbm.at[pl.ds(wid * pair_w, pair_w)], idx_v)
        pltpu.async_copy(o_hbm.at[idx_v], rows_v, sem).wait()

        def row_body(t, carry):
            for j in range(vpr):
                sl = pl.ds(j * _L, _L)
                rows_v[t, sl] = rows_v[2 * t, sl] + rows_v[2 * t + 1, sl]
            return carry

        lax.fori_loop(0, tok_w, row_body, 0)
        pltpu.sync_copy(rows_v.at[pl.ds(0, tok_w)],
                        out_hbm.at[pl.ds(wid * tok_w, tok_w)])

    return k(o_padded, pos_pair)


def _gemm_body(be_ref, nact_ref, x_ref, w13_ref, down_ref, pw_ref, o_ref):
    g = pl.program_id(0)

    @pl.when(g < nact_ref[0])
    def _():
        x = x_ref[...]
        w13 = w13_ref[0]
        h = jax.lax.dot_general(
            x, w13, (((1,), (1,)), ((), ())),
            preferred_element_type=jnp.float32)
        gate = h[:, :_INTER]
        up = h[:, _INTER:]
        act = gate * jax.nn.sigmoid(gate) * up
        dn = down_ref[0]
        o = jax.lax.dot_general(
            act, dn, (((1,), (1,)), ((), ())),
            preferred_element_type=jnp.float32)
        o_ref[...] = o * pw_ref[...]


def _grouped_gemm(x_padded, w13, down_proj, pw_padded, block_expert,
                  num_active):
    G = block_expert.shape[0]
    H, I = _HIDDEN, _INTER
    grid_spec = pltpu.PrefetchScalarGridSpec(
        num_scalar_prefetch=2,
        grid=(G,),
        in_specs=[
            pl.BlockSpec((_BM, H), lambda g, be, na: (g, 0)),
            pl.BlockSpec((1, 2 * I, H), lambda g, be, na: (be[g], 0, 0)),
            pl.BlockSpec((1, H, I), lambda g, be, na: (be[g], 0, 0)),
            pl.BlockSpec((_BM, 1), lambda g, be, na: (g, 0)),
        ],
        out_specs=pl.BlockSpec((_BM, H), lambda g, be, na: (g, 0)),
    )
    return pl.pallas_call(
        _gemm_body,
        grid_spec=grid_spec,
        out_shape=jax.ShapeDtypeStruct((G * _BM, H), jnp.float32),
    )(block_expert, num_active, x_padded, w13, down_proj, pw_padded)


@jax.jit
def _moe(x, expert_indices, expert_weights, w13, down_proj):
    B, S, H = x.shape
    E = w13.shape[0]
    top_k = expert_indices.shape[-1]
    N = B * S
    P = N * top_k
    G = P // _BM + E
    P_pad = G * _BM

    x_flat = x.reshape(N, H)
    flat_e = expert_indices.reshape(P)
    w_flat = expert_weights.reshape(P)

    order = jnp.argsort(flat_e)
    e_sorted = flat_e[order]
    sizes = jnp.bincount(flat_e, length=E)
    blocks_e = (sizes + _BM - 1) // _BM
    starts_unpad = jnp.cumsum(sizes) - sizes
    block_start_e = jnp.cumsum(blocks_e) - blocks_e
    ranks = jnp.arange(P, dtype=jnp.int32) - starts_unpad[e_sorted]
    pos_sorted = (block_start_e[e_sorted] * _BM + ranks).astype(jnp.int32)

    tok_sorted = (order // top_k).astype(jnp.int32)
    pw_padded = jnp.zeros((P_pad, 1), jnp.float32).at[pos_sorted, 0].set(
        w_flat[order])
    pos_pair = jnp.zeros((P,), jnp.int32).at[order].set(pos_sorted)

    num_active = jnp.sum(blocks_e).astype(jnp.int32).reshape(1)
    block_expert = jnp.minimum(
        jnp.searchsorted(jnp.cumsum(blocks_e), jnp.arange(G), side="right"),
        E - 1).astype(jnp.int32)

    x_padded = _sc_dispatch(x_flat, tok_sorted, pos_sorted, P_pad)

    o_padded = _grouped_gemm(x_padded, w13, down_proj, pw_padded,
                             block_expert, num_active)

    out = _sc_combine(o_padded, pos_pair, N)
    return out.reshape(B, S, H)


def kernel(x, expert_indices, expert_weights, w13, down_proj):
    return _moe(x, expert_indices, expert_weights, w13, down_proj)


# speedup vs baseline: 1.6703x; 1.1128x over previous
import functools

import jax
import jax.numpy as jnp
from jax import lax
from jax.experimental import pallas as pl
from jax.experimental.pallas import tpu as pltpu
from jax.experimental.pallas import tpu_sc as plsc

_HIDDEN = 768
_INTER = 2048
_BM = 128
_NC = 2
_NS = 16
_NW = _NC * _NS
_L = 16


def _sc_dispatch(x_flat, tok_sorted, pos_sorted, p_pad):
    N, H = x_flat.shape
    P = tok_sorted.shape[0]
    per_w = P // _NW
    mesh = plsc.VectorSubcoreMesh(core_axis_name="c", subcore_axis_name="s")

    @functools.partial(
        pl.kernel, mesh=mesh,
        out_type=jax.ShapeDtypeStruct((p_pad, H), jnp.float32),
        scratch_types=[
            pltpu.VMEM((per_w,), jnp.int32),
            pltpu.VMEM((per_w,), jnp.int32),
            pltpu.VMEM((per_w, H), jnp.float32),
            pltpu.SemaphoreType.DMA,
            pltpu.SemaphoreType.DMA,
        ],
    )
    def k(x_hbm, tok_hbm, pos_hbm, out_hbm, tok_v, pos_v, rows_v, sem_g,
          sem_s):
        wid = lax.axis_index("s") * _NC + lax.axis_index("c")
        base = wid * per_w
        pltpu.sync_copy(tok_hbm.at[pl.ds(base, per_w)], tok_v)
        pltpu.sync_copy(pos_hbm.at[pl.ds(base, per_w)], pos_v)
        pltpu.async_copy(x_hbm.at[tok_v], rows_v, sem_g).wait()
        pltpu.async_copy(rows_v, out_hbm.at[pos_v], sem_s).wait()

    return k(x_flat, tok_sorted, pos_sorted)


def _sc_combine(o_padded, pos_even, pos_odd, w_even, w_odd, n_tokens):
    H = o_padded.shape[1]
    tok_w = n_tokens // _NW
    mesh = plsc.VectorSubcoreMesh(core_axis_name="c", subcore_axis_name="s")

    vpr = H // _L

    @functools.partial(
        pl.kernel, mesh=mesh,
        out_type=jax.ShapeDtypeStruct((n_tokens, H), jnp.float32),
        scratch_types=[
            pltpu.VMEM((tok_w,), jnp.int32),
            pltpu.VMEM((tok_w,), jnp.int32),
            pltpu.VMEM((tok_w + _L,), jnp.float32),
            pltpu.VMEM((tok_w + _L,), jnp.float32),
            pltpu.VMEM((tok_w, H), jnp.float32),
            pltpu.VMEM((tok_w, H), jnp.float32),
            pltpu.SemaphoreType.DMA,
            pltpu.SemaphoreType.DMA,
        ],
    )
    def k(o_hbm, pe_hbm, po_hbm, we_hbm, wo_hbm, out_hbm, idx0_v, idx1_v,
          w0_v, w1_v, r0_v, r1_v, sem0, sem1):
        wid = lax.axis_index("s") * _NC + lax.axis_index("c")
        base = wid * tok_w
        pltpu.sync_copy(pe_hbm.at[pl.ds(base, tok_w)], idx0_v)
        pltpu.sync_copy(po_hbm.at[pl.ds(base, tok_w)], idx1_v)
        g0 = pltpu.async_copy(o_hbm.at[idx0_v], r0_v, sem0)
        g1 = pltpu.async_copy(o_hbm.at[idx1_v], r1_v, sem1)
        pltpu.sync_copy(we_hbm.at[pl.ds(base, tok_w)],
                        w0_v.at[pl.ds(0, tok_w)])
        pltpu.sync_copy(wo_hbm.at[pl.ds(base, tok_w)],
                        w1_v.at[pl.ds(0, tok_w)])
        g0.wait()
        g1.wait()

        def row_body(t, carry):
            wv0 = jnp.zeros((_L,), jnp.float32) + w0_v[pl.ds(t, _L)][0]
            wv1 = jnp.zeros((_L,), jnp.float32) + w1_v[pl.ds(t, _L)][0]
            for j in range(vpr):
                sl = pl.ds(j * _L, _L)
                r0_v[t, sl] = r0_v[t, sl] * wv0 + r1_v[t, sl] * wv1
            return carry

        lax.fori_loop(0, tok_w, row_body, 0)
        pltpu.sync_copy(r0_v, out_hbm.at[pl.ds(base, tok_w)])

    return k(o_padded, pos_even, pos_odd, w_even, w_odd)


def _gemm_body(be_ref, nact_ref, x_ref, w13_ref, down_ref, o_ref):
    g = pl.program_id(0)

    @pl.when(g < nact_ref[0])
    def _():
        x = x_ref[...]
        w13 = w13_ref[0]
        h = jax.lax.dot_general(
            x, w13, (((1,), (1,)), ((), ())),
            preferred_element_type=jnp.float32,
            precision=jax.lax.Precision.DEFAULT)
        gate = h[:, :_INTER]
        up = h[:, _INTER:]
        act = gate * jax.nn.sigmoid(gate) * up
        dn = down_ref[0]
        o = jax.lax.dot_general(
            act, dn, (((1,), (1,)), ((), ())),
            preferred_element_type=jnp.float32,
            precision=jax.lax.Precision.DEFAULT)
        o_ref[...] = o


def _grouped_gemm(x_padded, w13, down_proj, block_expert, num_active):
    G = block_expert.shape[0]
    H, I = _HIDDEN, _INTER
    grid_spec = pltpu.PrefetchScalarGridSpec(
        num_scalar_prefetch=2,
        grid=(G,),
        in_specs=[
            pl.BlockSpec((_BM, H),
                         lambda g, be, na: (jnp.minimum(g, na[0] - 1), 0)),
            pl.BlockSpec((1, 2 * I, H), lambda g, be, na: (be[g], 0, 0)),
            pl.BlockSpec((1, H, I), lambda g, be, na: (be[g], 0, 0)),
        ],
        out_specs=pl.BlockSpec(
            (_BM, H), lambda g, be, na: (jnp.minimum(g, na[0] - 1), 0)),
    )
    return pl.pallas_call(
        _gemm_body,
        grid_spec=grid_spec,
        out_shape=jax.ShapeDtypeStruct((G * _BM, H), jnp.float32),
    )(block_expert, num_active, x_padded, w13, down_proj)


@jax.jit
def _moe(x, expert_indices, expert_weights, w13, down_proj):
    B, S, H = x.shape
    E = w13.shape[0]
    top_k = expert_indices.shape[-1]
    N = B * S
    P = N * top_k
    G = P // _BM + E
    P_pad = G * _BM

    x_flat = x.reshape(N, H)
    flat_e = expert_indices.reshape(P)
    w_flat = expert_weights.reshape(P)

    order = jnp.argsort(flat_e)
    e_sorted = flat_e[order]
    sizes = jnp.bincount(flat_e, length=E)
    blocks_e = (sizes + _BM - 1) // _BM
    starts_unpad = jnp.cumsum(sizes) - sizes
    block_start_e = jnp.cumsum(blocks_e) - blocks_e
    ranks = jnp.arange(P, dtype=jnp.int32) - starts_unpad[e_sorted]
    pos_sorted = (block_start_e[e_sorted] * _BM + ranks).astype(jnp.int32)

    tok_sorted = (order // top_k).astype(jnp.int32)
    pos_pair = jnp.zeros((P,), jnp.int32).at[order].set(pos_sorted)

    num_active = jnp.sum(blocks_e).astype(jnp.int32).reshape(1)
    block_expert = jnp.minimum(
        jnp.searchsorted(jnp.cumsum(blocks_e), jnp.arange(G), side="right"),
        E - 1).astype(jnp.int32)

    x_padded = _sc_dispatch(x_flat, tok_sorted, pos_sorted, P_pad)

    o_padded = _grouped_gemm(x_padded, w13, down_proj, block_expert,
                             num_active)

    out = _sc_combine(o_padded, pos_pair[0::2], pos_pair[1::2],
                      w_flat[0::2], w_flat[1::2], N)
    return out.reshape(B, S, H)


def kernel(x, expert_indices, expert_weights, w13, down_proj):
    return _moe(x, expert_indices, expert_weights, w13, down_proj)
